# baseline (speedup 1.0000x reference)
import functools

import jax
import jax.numpy as jnp
from jax import lax
from jax.experimental import pallas as pl
from jax.experimental.pallas import tpu as pltpu
from jax.experimental.pallas import tpu_sc as plsc

NC = 2
NS = 16
NW = NC * NS
L = 16


def _sc_fact_bank(batch, dim, num_facts, chunk):
    rows_per_w = batch // NW
    n_chunks = num_facts // chunk
    n_vec = chunk // L
    mesh = plsc.VectorSubcoreMesh(core_axis_name="c", subcore_axis_name="s")

    @functools.partial(
        pl.kernel,
        mesh=mesh,
        out_type=jax.ShapeDtypeStruct((batch * num_facts,), jnp.float32),
        compiler_params=pltpu.CompilerParams(needs_layout_passes=False),
        scratch_types=[
            pltpu.VMEM((rows_per_w * dim,), jnp.float32),
            pltpu.VMEM((chunk,), jnp.int32),
            pltpu.VMEM((chunk,), jnp.int32),
            pltpu.VMEM((chunk,), jnp.float32),
            pltpu.VMEM((chunk,), jnp.float32),
            pltpu.VMEM((chunk,), jnp.float32),
            pltpu.VMEM((chunk,), jnp.float32),
            pltpu.VMEM((chunk,), jnp.float32),
        ],
    )
    def k(x_hbm, th_hbm, lk_hbm, ii_hbm, jj_hbm, out_hbm,
          xflat, iv, jv, thv, lkv, nkv, av, outb):
        wid = lax.axis_index("s") * NC + lax.axis_index("c")
        base = wid * rows_per_w
        pltpu.sync_copy(x_hbm.at[pl.ds(base * dim, rows_per_w * dim)], xflat)

        for c in range(n_chunks):
            off = c * chunk
            pltpu.sync_copy(ii_hbm.at[pl.ds(off, chunk)], iv)
            pltpu.sync_copy(jj_hbm.at[pl.ds(off, chunk)], jv)
            pltpu.sync_copy(th_hbm.at[pl.ds(off, chunk)], thv)
            pltpu.sync_copy(lk_hbm.at[pl.ds(off, chunk)], lkv)

            def pre(v, _):
                s = pl.ds(v * L, L)
                kap = jnp.clip(jnp.exp(lkv[s]), 0.5, 50.0)
                nkv[s] = -kap
                av[s] = kap * thv[s]
                return _

            lax.fori_loop(0, n_vec, pre, None)

            def row_body(r, _):
                rb = jnp.full((L,), r * dim, dtype=jnp.int32)

                def vec_body(v, _):
                    s = pl.ds(v * L, L)
                    xi = plsc.load_gather(xflat, [rb + iv[s]])
                    xj = plsc.load_gather(xflat, [rb + jv[s]])
                    e = jnp.exp(nkv[s] * (xi - xj) + av[s])
                    outb[s] = 1.0 / (1.0 + e)
                    return _

                lax.fori_loop(0, n_vec, vec_body, None)
                pltpu.sync_copy(
                    outb,
                    out_hbm.at[pl.ds((base + r) * num_facts + off, chunk)])
                return _

            lax.fori_loop(0, rows_per_w, row_body, None)

    return k


def kernel(x, th, log_kappa, i_idx, j_idx):
    batch, dim = x.shape
    num_facts = i_idx.shape[0]
    chunk = 2176
    k = _sc_fact_bank(batch, dim, num_facts, chunk)
    out = k(x.reshape(-1), th, log_kappa, i_idx, j_idx)
    return out.reshape(batch, num_facts)


# speedup vs baseline: 1.0534x; 1.0534x over previous
import functools

import jax
import jax.numpy as jnp
from jax import lax
from jax.experimental import pallas as pl
from jax.experimental.pallas import tpu as pltpu
from jax.experimental.pallas import tpu_sc as plsc

NC = 2
NS = 16
NW = NC * NS
L = 16


def _sc_fact_bank(batch, dim, num_facts, chunk, rg):
    rows_per_w = batch // NW
    n_chunks = num_facts // chunk
    n_vec = chunk // L
    n_groups = rows_per_w // rg
    assert n_groups == 2
    mesh = plsc.VectorSubcoreMesh(core_axis_name="c", subcore_axis_name="s")

    @functools.partial(
        pl.kernel,
        mesh=mesh,
        out_type=jax.ShapeDtypeStruct((batch * num_facts,), jnp.float32),
        compiler_params=pltpu.CompilerParams(needs_layout_passes=False),
        scratch_types=[
            pltpu.VMEM((rows_per_w * dim,), jnp.float32),
            pltpu.VMEM((chunk,), jnp.int32),
            pltpu.VMEM((chunk,), jnp.int32),
            pltpu.VMEM((chunk,), jnp.float32),
            pltpu.VMEM((chunk,), jnp.float32),
            pltpu.VMEM((rg * chunk,), jnp.float32),
            pltpu.VMEM((rg * chunk,), jnp.float32),
            pltpu.SemaphoreType.DMA,
            pltpu.SemaphoreType.DMA,
        ],
    )
    def k(x_hbm, th_hbm, lk_hbm, ii_hbm, jj_hbm, out_hbm,
          xflat, iv, jv, nkv, av, ob0, ob1, sem0, sem1):
        wid = lax.axis_index("s") * NC + lax.axis_index("c")
        base = wid * rows_per_w
        pltpu.sync_copy(x_hbm.at[pl.ds(base * dim, rows_per_w * dim)], xflat)

        def drain(ob, sem):
            for q in range(rg):
                pltpu.make_async_copy(
                    ob.at[pl.ds(q * chunk, chunk)],
                    out_hbm.at[pl.ds(q * chunk, chunk)], sem).wait()

        def compute_group(ob, row0):
            def vec_body(v, _):
                s = pl.ds(v * L, L)
                nk = nkv[s]
                a = av[s]
                fi = iv[s] + (row0 * dim)
                fj = jv[s] + (row0 * dim)
                for q in range(rg):
                    xi = plsc.load_gather(xflat, [fi])
                    xj = plsc.load_gather(xflat, [fj])
                    e = jnp.exp(nk * (xi - xj) + a)
                    ob[pl.ds(q * chunk + v * L, L)] = 1.0 / (1.0 + e)
                    if q != rg - 1:
                        fi = fi + dim
                        fj = fj + dim
                return _

            lax.fori_loop(0, n_vec, vec_body, None)

        def fire(ob, sem, row0, off):
            for q in range(rg):
                dst = pl.ds((base + row0 + q) * num_facts + off, chunk)
                pltpu.async_copy(
                    ob.at[pl.ds(q * chunk, chunk)], out_hbm.at[dst], sem)

        def chunk_body(c, _):
            off = pl.multiple_of(c * chunk, 256)
            pltpu.sync_copy(ii_hbm.at[pl.ds(off, chunk)], iv)
            pltpu.sync_copy(jj_hbm.at[pl.ds(off, chunk)], jv)
            pltpu.sync_copy(th_hbm.at[pl.ds(off, chunk)], nkv)
            pltpu.sync_copy(lk_hbm.at[pl.ds(off, chunk)], av)

            def pre(v, _):
                s = pl.ds(v * L, L)
                kap = jnp.clip(jnp.exp(av[s]), 0.5, 50.0)
                av[s] = kap * nkv[s]
                nkv[s] = -kap
                return _

            lax.fori_loop(0, n_vec, pre, None)

            @pl.when(c > 0)
            def _drain0():
                drain(ob0, sem0)

            compute_group(ob0, 0)
            fire(ob0, sem0, 0, off)

            @pl.when(c > 0)
            def _drain1():
                drain(ob1, sem1)

            compute_group(ob1, rg)
            fire(ob1, sem1, rg, off)
            return _

        lax.fori_loop(0, n_chunks, chunk_body, None)
        drain(ob0, sem0)
        drain(ob1, sem1)

    return k


def kernel(x, th, log_kappa, i_idx, j_idx):
    batch, dim = x.shape
    num_facts = i_idx.shape[0]
    chunk = 2176
    k = _sc_fact_bank(batch, dim, num_facts, chunk, rg=16)
    out = k(x.reshape(-1), th, log_kappa, i_idx, j_idx)
    return out.reshape(batch, num_facts)


# speedup vs baseline: 5.0216x; 4.7671x over previous
import functools

import jax
import jax.numpy as jnp
from jax import lax
from jax.experimental import pallas as pl
from jax.experimental.pallas import tpu as pltpu
from jax.experimental.pallas import tpu_sc as plsc

NC = 2
NS = 16
NW = NC * NS
L = 16


def _sc_fact_bank(batch, dim, num_facts, chunk, rg):
    rows_per_w = batch // NW
    n_chunks = num_facts // chunk
    n_vec = chunk // L
    n_groups = rows_per_w // rg
    assert n_groups == 2
    mesh = plsc.VectorSubcoreMesh(core_axis_name="c", subcore_axis_name="s")

    @functools.partial(
        pl.kernel,
        mesh=mesh,
        out_type=jax.ShapeDtypeStruct((batch * num_facts,), jnp.float32),
        compiler_params=pltpu.CompilerParams(needs_layout_passes=False),
        scratch_types=[
            pltpu.VMEM((rows_per_w * dim,), jnp.float32),
            pltpu.VMEM((chunk,), jnp.int32),
            pltpu.VMEM((chunk,), jnp.int32),
            pltpu.VMEM((chunk,), jnp.float32),
            pltpu.VMEM((chunk,), jnp.float32),
            pltpu.VMEM((rg * chunk,), jnp.float32),
            pltpu.VMEM((rg * chunk,), jnp.float32),
            pltpu.SemaphoreType.DMA,
            pltpu.SemaphoreType.DMA,
        ],
    )
    def k(x_hbm, th_hbm, lk_hbm, ii_hbm, jj_hbm, out_hbm,
          xflat, iv, jv, nkv, av, ob0, ob1, sem0, sem1):
        wid = lax.axis_index("s") * NC + lax.axis_index("c")
        base = wid * rows_per_w
        pltpu.sync_copy(x_hbm.at[pl.ds(base * dim, rows_per_w * dim)], xflat)

        def drain(ob, sem):
            for q in range(rg):
                pltpu.make_async_copy(
                    ob.at[pl.ds(q * chunk, chunk)],
                    out_hbm.at[pl.ds(q * chunk, chunk)], sem).wait()

        def compute_group(ob, row0):
            @plsc.parallel_loop(0, n_vec)
            def vec_body(v):
                s = pl.ds(v * L, L)
                nk = nkv[s]
                a = av[s]
                fi = iv[s] + (row0 * dim)
                fj = jv[s] + (row0 * dim)
                for q in range(rg):
                    xi = plsc.load_gather(xflat, [fi])
                    xj = plsc.load_gather(xflat, [fj])
                    e = jnp.exp(nk * (xi - xj) + a)
                    ob[pl.ds(q * chunk + v * L, L)] = 1.0 / (1.0 + e)
                    if q != rg - 1:
                        fi = fi + dim
                        fj = fj + dim

        def fire(ob, sem, row0, off):
            for q in range(rg):
                dst = pl.ds((base + row0 + q) * num_facts + off, chunk)
                pltpu.async_copy(
                    ob.at[pl.ds(q * chunk, chunk)], out_hbm.at[dst], sem)

        def chunk_body(c, _):
            off = pl.multiple_of(c * chunk, 256)
            pltpu.sync_copy(ii_hbm.at[pl.ds(off, chunk)], iv)
            pltpu.sync_copy(jj_hbm.at[pl.ds(off, chunk)], jv)
            pltpu.sync_copy(th_hbm.at[pl.ds(off, chunk)], nkv)
            pltpu.sync_copy(lk_hbm.at[pl.ds(off, chunk)], av)

            @plsc.parallel_loop(0, n_vec)
            def pre(v):
                s = pl.ds(v * L, L)
                kap = jnp.clip(jnp.exp(av[s]), 0.5, 50.0)
                av[s] = kap * nkv[s]
                nkv[s] = -kap

            @pl.when(c > 0)
            def _drain0():
                drain(ob0, sem0)

            compute_group(ob0, 0)
            fire(ob0, sem0, 0, off)

            @pl.when(c > 0)
            def _drain1():
                drain(ob1, sem1)

            compute_group(ob1, rg)
            fire(ob1, sem1, rg, off)
            return _

        lax.fori_loop(0, n_chunks, chunk_body, None)
        drain(ob0, sem0)
        drain(ob1, sem1)

    return k


def kernel(x, th, log_kappa, i_idx, j_idx):
    batch, dim = x.shape
    num_facts = i_idx.shape[0]
    chunk = 2176
    k = _sc_fact_bank(batch, dim, num_facts, chunk, rg=16)
    out = k(x.reshape(-1), th, log_kappa, i_idx, j_idx)
    return out.reshape(batch, num_facts)
